# Initial kernel scaffold; baseline (speedup 1.0000x reference)
import jax
import jax.numpy as jnp
from jax.experimental import pallas as pl


def kernel(nodes, hist_vc, hist_r, feat, r_table, W_agg, b_agg, W1, b1):
    raise NotImplementedError("write your pallas kernel here")


# speedup vs baseline: 3.8159x; 3.8159x over previous
import functools

import jax
import jax.numpy as jnp
from jax import lax
from jax.experimental import pallas as pl
from jax.experimental.pallas import tpu as pltpu
from jax.experimental.pallas import tpu_sc as plsc

N_NODES = 100000
D = 64
L = 50
B = 4096
NR = 5

NTILES = 32
BPT = B // NTILES
IPT = BPT * L
NCHUNK = L
CH = BPT


def _sc_gather_body(nodes_hbm, histvc_hbm, histr_hbm, feat_hbm,
                    item_out, r_out, self_out,
                    nodes_v, idx_v, items_v, r_v, selfbuf, rowa, rowb,
                    semg, sem1, sema, semb):
    c = lax.axis_index("c")
    s = lax.axis_index("s")
    wid = s * 2 + c
    base = wid * BPT

    pltpu.sync_copy(nodes_hbm.at[pl.ds(base, BPT)], nodes_v)

    iota = lax.iota(jnp.int32, 16)

    def build(j, _):
        for k in range(CH // 16):
            g = j * CH + k * 16 + iota
            i = g // L
            l = g - i * L
            nd = plsc.load_gather(nodes_v, [i])
            idx_v[j, pl.ds(k * 16, 16)] = nd * L + l
        return 0

    lax.fori_loop(0, NCHUNK, build, 0, unroll=False)

    def lvl1(j, _):
        ds = []
        for jj in range(10):
            row = j * 10 + jj
            ds.append(pltpu.async_copy(
                histvc_hbm.at[idx_v.at[row]], items_v.at[row], sem1))
            ds.append(pltpu.async_copy(
                histr_hbm.at[idx_v.at[row]], r_v.at[row], sem1))
        for d in ds:
            d.wait()
        return 0

    lax.fori_loop(0, NCHUNK // 10, lvl1, 0, unroll=False)

    pltpu.sync_copy(r_v, r_out.at[wid])

    pltpu.async_copy(feat_hbm.at[nodes_v], selfbuf, semg).wait()
    pltpu.sync_copy(selfbuf, self_out.at[wid])

    pltpu.async_copy(feat_hbm.at[items_v.at[0]], rowa, sema)
    pltpu.async_copy(feat_hbm.at[items_v.at[1]], rowb, semb)

    def lvl2(j, _):
        c0 = 2 * j
        pltpu.make_async_copy(feat_hbm.at[items_v.at[c0]], rowa, sema).wait()
        pltpu.sync_copy(rowa, item_out.at[wid, c0])

        @pl.when(c0 + 2 < NCHUNK)
        def _fire_a():
            pltpu.async_copy(feat_hbm.at[items_v.at[c0 + 2]], rowa, sema)

        pltpu.make_async_copy(
            feat_hbm.at[items_v.at[c0 + 1]], rowb, semb).wait()
        pltpu.sync_copy(rowb, item_out.at[wid, c0 + 1])

        @pl.when(c0 + 3 < NCHUNK)
        def _fire_b():
            pltpu.async_copy(feat_hbm.at[items_v.at[c0 + 3]], rowb, semb)

        return 0

    lax.fori_loop(0, NCHUNK // 2, lvl2, 0, unroll=False)


@jax.jit
def _sc_gather(nodes, histvc_flat, histr_flat, feat):
    mesh = plsc.VectorSubcoreMesh(core_axis_name="c", subcore_axis_name="s")
    f = functools.partial(
        pl.kernel,
        compiler_params=pltpu.CompilerParams(
            use_tc_tiling_on_sc=False, needs_layout_passes=False),
        out_type=(
            jax.ShapeDtypeStruct((NTILES, NCHUNK, CH, D), jnp.float32),
            jax.ShapeDtypeStruct((NTILES, NCHUNK, CH), jnp.int32),
            jax.ShapeDtypeStruct((NTILES, BPT, D), jnp.float32),
        ),
        mesh=mesh,
        scratch_types=[
            pltpu.VMEM((BPT,), jnp.int32),
            pltpu.VMEM((NCHUNK, CH), jnp.int32),
            pltpu.VMEM((NCHUNK, CH), jnp.int32),
            pltpu.VMEM((NCHUNK, CH), jnp.int32),
            pltpu.VMEM((BPT, D), jnp.float32),
            pltpu.VMEM((CH, D), jnp.float32),
            pltpu.VMEM((CH, D), jnp.float32),
            pltpu.SemaphoreType.DMA,
            pltpu.SemaphoreType.DMA,
            pltpu.SemaphoreType.DMA,
            pltpu.SemaphoreType.DMA,
        ],
    )(_sc_gather_body)
    return f(nodes, histvc_flat, histr_flat, feat)


def _tc_body(item_ref, r_ref, self_ref, wagg_ref, rtab_ref, bagg_ref,
             w1_ref, b1_ref, out_ref):
    wa_top = wagg_ref[:D, :]
    wa_bot = wagg_ref[D:, :]
    ctab = jnp.dot(rtab_ref[...], wa_bot,
                   preferred_element_type=jnp.float32)
    proj = jnp.dot(item_ref[...], wa_top,
                   preferred_element_type=jnp.float32)
    r = r_ref[...]
    crows = jnp.zeros((IPT, D), jnp.float32)
    for k in range(NR):
        crows = crows + (r == k).astype(jnp.float32) * ctab[k:k + 1, :]
    h = jnp.maximum(proj + crows + bagg_ref[...], 0.0)
    neigh = jnp.sum(h.reshape(BPT, L, D), axis=1) * (1.0 / L)
    out = jnp.maximum(
        jnp.dot(self_ref[...], w1_ref[:D, :],
                preferred_element_type=jnp.float32)
        + jnp.dot(neigh, w1_ref[D:, :], preferred_element_type=jnp.float32)
        + b1_ref[...], 0.0)
    out_ref[...] = out


@jax.jit
def _tc_compute(item_e, r2, self_f, w_agg, r_table, b_agg2, w1, b12):
    return pl.pallas_call(
        _tc_body,
        grid=(NTILES,),
        in_specs=[
            pl.BlockSpec((IPT, D), lambda j: (j, 0)),
            pl.BlockSpec((IPT, 1), lambda j: (j, 0)),
            pl.BlockSpec((BPT, D), lambda j: (j, 0)),
            pl.BlockSpec((2 * D, D), lambda j: (0, 0)),
            pl.BlockSpec((NR, D), lambda j: (0, 0)),
            pl.BlockSpec((1, D), lambda j: (0, 0)),
            pl.BlockSpec((2 * D, D), lambda j: (0, 0)),
            pl.BlockSpec((1, D), lambda j: (0, 0)),
        ],
        out_specs=pl.BlockSpec((BPT, D), lambda j: (j, 0)),
        out_shape=jax.ShapeDtypeStruct((B, D), jnp.float32),
    )(item_e, r2, self_f, w_agg, r_table, b_agg2, w1, b12)


def kernel(nodes, hist_vc, hist_r, feat, r_table, W_agg, b_agg, W1, b1):
    item_e, r_out, self_out = _sc_gather(
        nodes, hist_vc.reshape(-1), hist_r.reshape(-1), feat)
    return _tc_compute(
        item_e.reshape(B * L, D),
        r_out.reshape(B * L, 1),
        self_out.reshape(B, D),
        W_agg, r_table,
        b_agg.reshape(1, D),
        W1, b1.reshape(1, D))
